# baseline (speedup 1.0000x reference)
import functools

import jax
import jax.numpy as jnp
from jax import lax
from jax.experimental import pallas as pl
from jax.experimental.pallas import tpu as pltpu
from jax.experimental.pallas import tpu_sc as plsc

VOCAB = 100000
EMB = 64
BATCH = 16384

TBLK = 2048


def _tblock(t_ref, o_ref):
    o_ref[...] = t_ref[...].T


@jax.jit
def _transpose_tc(table_t):
    grid = (VOCAB + TBLK - 1) // TBLK
    return pl.pallas_call(
        _tblock,
        grid=(grid,),
        in_specs=[pl.BlockSpec((EMB, TBLK), lambda j: (0, j))],
        out_specs=pl.BlockSpec((TBLK, EMB), lambda j: (j, 0)),
        out_shape=jax.ShapeDtypeStruct((VOCAB, EMB), jnp.float32),
    )(table_t)


@jax.jit
def _gather_sc(table, idx):
    info = plsc.get_sparse_core_info()
    nw = info.num_cores * info.num_subcores
    b_per_w = BATCH // nw
    n_tc = b_per_w // 128
    mesh = plsc.VectorSubcoreMesh(core_axis_name="c", subcore_axis_name="s")

    @functools.partial(
        pl.kernel,
        mesh=mesh,
        out_type=jax.ShapeDtypeStruct((EMB // 8, BATCH // 128, 8, 128), jnp.float32),
        scratch_types=[
            pltpu.VMEM((b_per_w,), jnp.int32),
            pltpu.VMEM((b_per_w, EMB), jnp.float32),
            pltpu.VMEM((EMB, b_per_w), jnp.float32),
            [pltpu.SemaphoreType.DMA] * 4,
            pltpu.SemaphoreType.DMA,
        ],
        compiler_params=pltpu.CompilerParams(needs_layout_passes=False),
    )
    def k(table_hbm, idx_hbm, out4_hbm, idx_v, rows_v, out_t_v, sems, osem):
        wid = lax.axis_index("s") * info.num_cores + lax.axis_index("c")
        base = wid * b_per_w
        pltpu.sync_copy(idx_hbm.at[pl.ds(base, b_per_w)], idx_v)

        iota16 = lax.iota(jnp.int32, 16)
        sc_rows = b_per_w // 4

        for s in range(4):
            for c in range(sc_rows // 16):
                vec = idx_v[pl.ds(s * sc_rows + c * 16, 16)]
                for j in range(16):
                    r = vec[j]
                    pltpu.async_copy(
                        table_hbm.at[r],
                        rows_v.at[s * sc_rows + c * 16 + j],
                        sems[s],
                    )

        def extract(c):
            rows = iota16 + c * 16
            for d in range(EMB):
                vals = plsc.load_gather(
                    rows_v, [rows, jnp.full((16,), d, jnp.int32)]
                )
                out_t_v[d, pl.ds(c * 16, 16)] = vals

        for s in range(4):
            pltpu.make_async_copy(
                table_hbm.at[pl.ds(0, sc_rows)],
                rows_v.at[pl.ds(s * sc_rows, sc_rows)],
                sems[s],
            ).wait()

            def ebody(c, _):
                extract(c)
                return 0

            lax.fori_loop(
                s * (sc_rows // 16), (s + 1) * (sc_rows // 16), ebody, 0
            )

        for tcl in range(n_tc):
            for tr in range(EMB // 8):
                pltpu.async_copy(
                    out_t_v.at[pl.ds(tr * 8, 8), pl.ds(tcl * 128, 128)],
                    out4_hbm.at[tr, wid * n_tc + tcl],
                    osem,
                )
        for tcl in range(n_tc):
            for tr in range(EMB // 8):
                pltpu.make_async_copy(
                    out_t_v.at[pl.ds(0, 8), pl.ds(0, 128)],
                    out4_hbm.at[0, 0],
                    osem,
                ).wait()

    return k(table, idx)


def kernel(x, table):
    out4 = _gather_sc(table, x.astype(jnp.int32))
    return out4.transpose(0, 2, 1, 3).reshape(EMB, BATCH).T


# speedup vs baseline: 1.1690x; 1.1690x over previous
import functools

import jax
import jax.numpy as jnp
from jax import lax
from jax.experimental import pallas as pl
from jax.experimental.pallas import tpu as pltpu
from jax.experimental.pallas import tpu_sc as plsc

VOCAB = 100000
EMB = 64
BATCH = 16384

TBLK = 2048


def _tblock(t_ref, o_ref):
    o_ref[...] = t_ref[...].T


@jax.jit
def _transpose_tc(table_t):
    grid = (VOCAB + TBLK - 1) // TBLK
    return pl.pallas_call(
        _tblock,
        grid=(grid,),
        in_specs=[pl.BlockSpec((EMB, TBLK), lambda j: (0, j))],
        out_specs=pl.BlockSpec((TBLK, EMB), lambda j: (j, 0)),
        out_shape=jax.ShapeDtypeStruct((VOCAB, EMB), jnp.float32),
    )(table_t)


@jax.jit
def _gather_sc(table, idx):
    info = plsc.get_sparse_core_info()
    nw = info.num_cores * info.num_subcores
    b_per_w = BATCH // nw
    n_tc = b_per_w // 128
    mesh = plsc.VectorSubcoreMesh(core_axis_name="c", subcore_axis_name="s")

    @functools.partial(
        pl.kernel,
        mesh=mesh,
        out_type=jax.ShapeDtypeStruct((EMB // 8, BATCH // 128, 8, 128), jnp.float32),
        scratch_types=[
            pltpu.VMEM((b_per_w,), jnp.int32),
            pltpu.VMEM((b_per_w, EMB), jnp.float32),
            pltpu.VMEM((EMB, b_per_w), jnp.float32),
            [pltpu.SemaphoreType.DMA] * 4,
            pltpu.SemaphoreType.DMA,
        ],
        compiler_params=pltpu.CompilerParams(needs_layout_passes=False),
    )
    def k(table_hbm, idx_hbm, out4_hbm, idx_v, rows_v, out_t_v, sems, osem):
        wid = lax.axis_index("s") * info.num_cores + lax.axis_index("c")
        base = wid * b_per_w
        pltpu.sync_copy(idx_hbm.at[pl.ds(base, b_per_w)], idx_v)

        iota16 = lax.iota(jnp.int32, 16)
        sc_rows = b_per_w // 4

        for s in range(4):
            for c in range(sc_rows // 16):
                vec = idx_v[pl.ds(s * sc_rows + c * 16, 16)]
                for j in range(16):
                    r = vec[j]
                    pltpu.async_copy(
                        table_hbm.at[r],
                        rows_v.at[s * sc_rows + c * 16 + j],
                        sems[s],
                    )

        def extract(c):
            rows = iota16 + c * 16
            for d0 in range(0, EMB, 8):
                vals = [
                    plsc.load_gather(
                        rows_v, [rows, jnp.full((16,), d0 + t, jnp.int32)]
                    )
                    for t in range(8)
                ]
                for t in range(8):
                    out_t_v[d0 + t, pl.ds(c * 16, 16)] = vals[t]

        for s in range(4):
            pltpu.make_async_copy(
                table_hbm.at[pl.ds(0, sc_rows)],
                rows_v.at[pl.ds(s * sc_rows, sc_rows)],
                sems[s],
            ).wait()

            def ebody(c, _):
                extract(c)
                return 0

            lax.fori_loop(
                s * (sc_rows // 16), (s + 1) * (sc_rows // 16), ebody, 0
            )

        for tcl in range(n_tc):
            for tr in range(EMB // 8):
                pltpu.async_copy(
                    out_t_v.at[pl.ds(tr * 8, 8), pl.ds(tcl * 128, 128)],
                    out4_hbm.at[tr, wid * n_tc + tcl],
                    osem,
                )
        for tcl in range(n_tc):
            for tr in range(EMB // 8):
                pltpu.make_async_copy(
                    out_t_v.at[pl.ds(0, 8), pl.ds(0, 128)],
                    out4_hbm.at[0, 0],
                    osem,
                ).wait()

    return k(table, idx)


def kernel(x, table):
    out4 = _gather_sc(table, x.astype(jnp.int32))
    return out4.transpose(0, 2, 1, 3).reshape(EMB, BATCH).T


# speedup vs baseline: 1.2955x; 1.1082x over previous
import functools

import jax
import jax.numpy as jnp
from jax import lax
from jax.experimental import pallas as pl
from jax.experimental.pallas import tpu as pltpu
from jax.experimental.pallas import tpu_sc as plsc

VOCAB = 100000
EMB = 64
BATCH = 16384


@jax.jit
def _gather_sc(table, idx):
    info = plsc.get_sparse_core_info()
    nw = info.num_cores * info.num_subcores
    b_per_w = BATCH // nw
    mesh = plsc.VectorSubcoreMesh(core_axis_name="c", subcore_axis_name="s")

    @functools.partial(
        pl.kernel,
        mesh=mesh,
        out_type=jax.ShapeDtypeStruct((BATCH, EMB), jnp.float32),
        scratch_types=[
            pltpu.VMEM((b_per_w,), jnp.int32),
            pltpu.VMEM((b_per_w, EMB), jnp.float32),
            pltpu.SemaphoreType.DMA,
        ],
    )
    def k(table_hbm, idx_hbm, out_hbm, idx_v, rows_v, sem):
        wid = lax.axis_index("s") * info.num_cores + lax.axis_index("c")
        base = wid * b_per_w
        pltpu.sync_copy(idx_hbm.at[pl.ds(base, b_per_w)], idx_v)

        def body(c, _):
            vec = idx_v[pl.ds(c * 16, 16)]
            for j in range(16):
                r = vec[j]
                pltpu.async_copy(table_hbm.at[r], rows_v.at[c * 16 + j], sem)
            return 0

        lax.fori_loop(0, b_per_w // 16, body, 0)
        pltpu.make_async_copy(
            table_hbm.at[pl.ds(0, b_per_w)], rows_v, sem
        ).wait()
        pltpu.sync_copy(rows_v, out_hbm.at[pl.ds(base, b_per_w)])

    return k(table, idx)


def kernel(x, table):
    return _gather_sc(table, x.astype(jnp.int32))
